# baseline (speedup 1.0000x reference)
import functools

import jax
import jax.numpy as jnp
from jax import lax
from jax.experimental import pallas as pl
from jax.experimental.pallas import tpu as pltpu
from jax.experimental.pallas import tpu_sc as plsc

MAXP = 100000
D = 64
DP = 128
B_TOTAL = 4096 * 200
NC, NS, L = 2, 16, 16
NW = NC * NS
B_PER_W = B_TOTAL // NW
IDXW = 128
NSTREAM = 5
HALF = NSTREAM * IDXW
CHUNK = 2 * HALF
NIDX = 2 * NSTREAM
NCHUNK = B_PER_W // CHUNK
IROWS_PER_W = B_PER_W // IDXW


def _posenc_body(t2, table, out, idx_v, rows_v, isems, gsems, osems):
    wid = lax.axis_index("s") * NC + lax.axis_index("c")
    base = wid * B_PER_W
    irow0 = wid * IROWS_PER_W

    def idx_copy(ci, s):
        return pltpu.make_async_copy(
            t2.at[pl.ds(irow0 + ci * NIDX, NIDX)], idx_v.at[s], isems[s]
        )

    def adjust(s):
        for j in range(NIDX):
            for i in range(IDXW // L):
                v = idx_v[s, j, pl.ds(i * L, L)]
                idx_v[s, j, pl.ds(i * L, L)] = jnp.where(
                    v == 0, MAXP - 1, v - 1
                )

    def out_copy(ci, h):
        return pltpu.make_async_copy(
            rows_v.at[h],
            out.at[pl.ds(base + ci * CHUNK + h * HALF, HALF), pl.ds(0, D)],
            osems[h],
        )

    def fire_gathers(s, h):
        return [
            pltpu.async_copy(
                table.at[idx_v.at[s, h * NSTREAM + j]],
                rows_v.at[h, pl.ds(j * IDXW, IDXW)],
                gsems[h],
            )
            for j in range(NSTREAM)
        ]

    def maybe(pred, fn):
        if pred is True:
            fn()
        else:
            pl.when(pred)(fn)

    def half(ci, s, not_first, do_next_adjust, do_next_fire):
        maybe(not_first, lambda: out_copy(ci - 1, 0).wait())
        h0 = fire_gathers(s, 0)
        maybe(not_first, lambda: out_copy(ci - 1, 1).wait())
        h1 = fire_gathers(s, 1)

        def _stage():
            idx_copy(ci + 1, 1 - s).wait()
            adjust(1 - s)

        maybe(do_next_adjust, _stage)

        for hd in h0:
            hd.wait()
        out_copy(ci, 0).start()
        for hd in h1:
            hd.wait()

        maybe(do_next_fire, lambda: idx_copy(ci + 2, s).start())
        out_copy(ci, 1).start()

    def body(k, carry):
        not_last = k < NCHUNK // 2 - 1
        half(2 * k, 0, k > 0, True, not_last)
        half(2 * k + 1, 1, True, not_last, not_last)
        return carry

    idx_copy(0, 0).start()
    idx_copy(1, 1).start()
    idx_copy(0, 0).wait()
    adjust(0)
    lax.fori_loop(0, NCHUNK // 2, body, 0)
    out_copy(NCHUNK - 1, 0).wait()
    out_copy(NCHUNK - 1, 1).wait()


_posenc_call = functools.partial(
    pl.kernel,
    mesh=plsc.VectorSubcoreMesh(core_axis_name="c", subcore_axis_name="s"),
    out_type=jax.ShapeDtypeStruct((B_TOTAL, DP), jnp.float32),
    scratch_types=[
        pltpu.VMEM((2, NIDX, IDXW), jnp.int32),
        pltpu.VMEM((2, HALF, D), jnp.float32),
        [pltpu.SemaphoreType.DMA] * 2,
        [pltpu.SemaphoreType.DMA] * 2,
        [pltpu.SemaphoreType.DMA] * 2,
    ],
    compiler_params=pltpu.CompilerParams(use_tc_tiling_on_sc=False),
)(_posenc_body)


@jax.jit
def kernel(t, pos_enc):
    t2 = t.reshape(B_TOTAL // IDXW, IDXW)
    out = _posenc_call(t2, pos_enc)
    return out[:, :D].reshape(t.shape + (D,))
